# baseline (speedup 1.0000x reference)
import jax
import jax.numpy as jnp
from jax.experimental import pallas as pl
from jax.experimental.pallas import tpu as pltpu

_NEG = -2.3819763e+38
_B = 512


def _mask_kernel(o_ref, z_ref, n_ref, d_ref, sems):
    s = o_ref.shape[2]
    nblk = s // _B
    z_ref[...] = jnp.zeros((_B, s), jnp.float32)
    n_ref[...] = jnp.full((_B, s), _NEG, jnp.float32)
    rows = jax.lax.broadcasted_iota(jnp.int32, (_B, _B), 0)
    cols = jax.lax.broadcasted_iota(jnp.int32, (_B, _B), 1)
    d_ref[...] = jnp.where(cols > rows, _NEG, 0.0).astype(jnp.float32)

    copies = []
    for k in range(nblk):
        r0 = k * _B
        c0 = k * _B
        copies.append(
            (d_ref.at[:, :], o_ref.at[0, 0, pl.ds(r0, _B), pl.ds(c0, _B)])
        )
        if c0 + _B < s:
            w = s - (c0 + _B)
            copies.append(
                (n_ref.at[:, pl.ds(0, w)], o_ref.at[0, 0, pl.ds(r0, _B), pl.ds(c0 + _B, w)])
            )
        if c0 > 0:
            copies.append(
                (z_ref.at[:, pl.ds(0, c0)], o_ref.at[0, 0, pl.ds(r0, _B), pl.ds(0, c0)])
            )
    for idx, (src, dst) in enumerate(copies):
        pltpu.make_async_copy(src, dst, sems.at[idx]).start()
    for idx, (src, dst) in enumerate(copies):
        pltpu.make_async_copy(src, dst, sems.at[idx]).wait()


def kernel(forward, batch_size, data, device, temperature, top_p, top_k, kv_caches, output_len, is_str_prompt):
    S = data.shape[-2]
    n_copies = 3 * (S // _B) - 2
    return pl.pallas_call(
        _mask_kernel,
        out_specs=pl.BlockSpec(memory_space=pl.ANY),
        out_shape=jax.ShapeDtypeStruct((1, 1, S, S), jnp.float32),
        scratch_shapes=[
            pltpu.VMEM((_B, S), jnp.float32),
            pltpu.VMEM((_B, S), jnp.float32),
            pltpu.VMEM((_B, _B), jnp.float32),
            pltpu.SemaphoreType.DMA((n_copies,)),
        ],
    )()


# speedup vs baseline: 1.0961x; 1.0961x over previous
import jax
import jax.numpy as jnp
from jax.experimental import pallas as pl
from jax.experimental.pallas import tpu as pltpu

_NEG = -2.3819763e+38


def _chunks_for(s):
    chunks, rem = [], s
    for c in (32, 32, 64, 128, 256):
        if rem <= 0:
            break
        c = min(c, rem)
        chunks.append(c)
        rem -= c
    while rem > 0:
        c = min(512, rem)
        chunks.append(c)
        rem -= c
    return tuple(chunks)


def _mask_kernel(o_ref, scratch, sems):
    s = scratch.shape[1]
    base = 0
    for k, br in enumerate(_chunks_for(scratch.shape[0])):
        rows = jax.lax.broadcasted_iota(jnp.int32, (br, s), 0) + base
        cols = jax.lax.broadcasted_iota(jnp.int32, (br, s), 1)
        scratch[pl.ds(base, br), :] = jnp.where(cols > rows, _NEG, 0.0).astype(
            jnp.float32
        )
        pltpu.make_async_copy(
            scratch.at[pl.ds(base, br), :],
            o_ref.at[0, 0, pl.ds(base, br), :],
            sems.at[k],
        ).start()
        base += br
    base = 0
    for k, br in enumerate(_chunks_for(scratch.shape[0])):
        pltpu.make_async_copy(
            scratch.at[pl.ds(base, br), :],
            o_ref.at[0, 0, pl.ds(base, br), :],
            sems.at[k],
        ).wait()
        base += br


def kernel(forward, batch_size, data, device, temperature, top_p, top_k, kv_caches, output_len, is_str_prompt):
    S = data.shape[-2]
    return pl.pallas_call(
        _mask_kernel,
        out_specs=pl.BlockSpec(memory_space=pl.ANY),
        out_shape=jax.ShapeDtypeStruct((1, 1, S, S), jnp.float32),
        scratch_shapes=[
            pltpu.VMEM((S, S), jnp.float32),
            pltpu.SemaphoreType.DMA((len(_chunks_for(S)),)),
        ],
    )()
